# baseline (speedup 1.0000x reference)
import jax
import jax.numpy as jnp
import numpy as np
from jax.experimental import pallas as pl

_F = 6600
_J = 22
_JB = 11
_TN = 40
_F_BLK = 3328
_NF = 2


def _fwd_kernel(fr_ref, flx_ref, fly_ref, t_ref,
                sig_ref, amax_ref, smax_ref, reax_ref, out_ref):
    fr = fr_ref[0, 0]
    x = fr[:, 1:2]
    y = fr[:, 2:3]
    vx = fr[:, 3:4]
    vy = fr[:, 4:5]
    ax = fr[:, 5:6]
    ay = fr[:, 6:7]

    sigma = sig_ref[0, 0]
    a_max = amax_ref[0, 0]
    s_max = smax_ref[0, 0]
    reax_t = reax_ref[0, 0]

    vxr = ax * reax_t + vx
    vyr = ay * reax_t + vy
    xr = x + vx * reax_t + 0.5 * ax * reax_t * reax_t
    yr = y + vy * reax_t + 0.5 * ay * reax_t * reax_t

    flx = flx_ref[0]
    fly = fly_ref[0]

    dx = flx - xr
    dy = fly - yr
    dmag = jnp.sqrt(dx * dx + dy * dy)
    s0 = jnp.clip((dx * vxr + dy * vyr) / dmag, -s_max, s_max)
    t_lt = (s_max - s0) / a_max
    d_lt = t_lt * (s0 + s_max) * 0.5
    soa = s0 / a_max
    t_lt = jnp.where(d_lt > dmag,
                     -soa + jnp.sqrt(soa * soa + 2.0 * dmag / a_max),
                     t_lt)
    d_lt = jnp.maximum(jnp.minimum(d_lt, dmag), 0.0)
    t_tot = reax_t + t_lt + (dmag - d_lt) / s_max

    kh = 0.5 * (jnp.pi / jnp.sqrt(3.0)) / sigma
    zh = kh * t_tot
    th = kh * t_ref[...]

    v = th.reshape(1, _TN, 1) - zh.reshape(_JB, 1, _F_BLK)
    out_ref[0] = 0.5 + 0.5 * jnp.tanh(v)


def kernel(frame, tti_sigma, a_max, s_max, reax_t):
    B = frame.shape[0]

    x = np.linspace(0.5, 119.5, 120, dtype=np.float32)
    y = np.linspace(-0.5, 53.5, 55, dtype=np.float32)
    y[0] = -0.2
    yy, xx = np.meshgrid(y, x, indexing='ij')
    pad = _NF * _F_BLK - _F
    flx = jnp.asarray(np.pad(xx.reshape(_F), (0, pad), mode='edge')
                      .reshape(_NF, 1, _F_BLK))
    fly = jnp.asarray(np.pad(yy.reshape(_F), (0, pad), mode='edge')
                      .reshape(_NF, 1, _F_BLK))

    T = np.linspace(0.1, 4.0, _TN, dtype=np.float32)
    tcol = jnp.asarray(T.reshape(_TN, 1))

    fr4 = frame.reshape(B, _J // _JB, _JB, frame.shape[-1])

    def s11(v):
        return jnp.asarray(v, jnp.float32).reshape(1, 1)

    out = pl.pallas_call(
        _fwd_kernel,
        grid=(B, _J // _JB, _NF),
        in_specs=[
            pl.BlockSpec((1, 1, _JB, 12), lambda b, j, f: (b, j, 0, 0)),
            pl.BlockSpec((1, 1, _F_BLK), lambda b, j, f: (f, 0, 0)),
            pl.BlockSpec((1, 1, _F_BLK), lambda b, j, f: (f, 0, 0)),
            pl.BlockSpec((_TN, 1), lambda b, j, f: (0, 0)),
            pl.BlockSpec((1, 1), lambda b, j, f: (0, 0)),
            pl.BlockSpec((1, 1), lambda b, j, f: (0, 0)),
            pl.BlockSpec((1, 1), lambda b, j, f: (0, 0)),
            pl.BlockSpec((1, 1), lambda b, j, f: (0, 0)),
        ],
        out_specs=pl.BlockSpec((1, _JB, _TN, _F_BLK),
                               lambda b, j, f: (b, j, 0, f)),
        out_shape=jax.ShapeDtypeStruct((B, _J, _TN, _F), jnp.float32),
    )(fr4, flx, fly, tcol,
      s11(tti_sigma), s11(a_max), s11(s_max), s11(reax_t))

    return out.transpose(0, 3, 2, 1)


# speedup vs baseline: 1.1783x; 1.1783x over previous
import jax
import jax.numpy as jnp
import numpy as np
from jax.experimental import pallas as pl

_F = 6600
_J = 22
_JB = 11
_TN = 40
_F_BLK = 6656
_NF = 1


def _fwd_kernel(fr_ref, flx_ref, fly_ref, t_ref,
                sig_ref, amax_ref, smax_ref, reax_ref, out_ref):
    fr = fr_ref[0, 0]
    x = fr[:, 1:2]
    y = fr[:, 2:3]
    vx = fr[:, 3:4]
    vy = fr[:, 4:5]
    ax = fr[:, 5:6]
    ay = fr[:, 6:7]

    sigma = sig_ref[0, 0]
    a_max = amax_ref[0, 0]
    s_max = smax_ref[0, 0]
    reax_t = reax_ref[0, 0]

    vxr = ax * reax_t + vx
    vyr = ay * reax_t + vy
    xr = x + vx * reax_t + 0.5 * ax * reax_t * reax_t
    yr = y + vy * reax_t + 0.5 * ay * reax_t * reax_t

    flx = flx_ref[0]
    fly = fly_ref[0]

    dx = flx - xr
    dy = fly - yr
    dmag = jnp.sqrt(dx * dx + dy * dy)
    s0 = jnp.clip((dx * vxr + dy * vyr) / dmag, -s_max, s_max)
    t_lt = (s_max - s0) / a_max
    d_lt = t_lt * (s0 + s_max) * 0.5
    soa = s0 / a_max
    t_lt = jnp.where(d_lt > dmag,
                     -soa + jnp.sqrt(soa * soa + 2.0 * dmag / a_max),
                     t_lt)
    d_lt = jnp.maximum(jnp.minimum(d_lt, dmag), 0.0)
    t_tot = reax_t + t_lt + (dmag - d_lt) / s_max

    kk = (jnp.pi / jnp.sqrt(3.0)) / sigma
    ez = jnp.exp(kk * t_tot)
    ct = jnp.exp(-kk * t_ref[...])

    w = ct.reshape(1, _TN, 1) * ez.reshape(_JB, 1, _F_BLK) + 1.0
    out_ref[0] = 1.0 / w


def kernel(frame, tti_sigma, a_max, s_max, reax_t):
    B = frame.shape[0]

    x = np.linspace(0.5, 119.5, 120, dtype=np.float32)
    y = np.linspace(-0.5, 53.5, 55, dtype=np.float32)
    y[0] = -0.2
    yy, xx = np.meshgrid(y, x, indexing='ij')
    pad = _NF * _F_BLK - _F
    flx = jnp.asarray(np.pad(xx.reshape(_F), (0, pad), mode='edge')
                      .reshape(_NF, 1, _F_BLK))
    fly = jnp.asarray(np.pad(yy.reshape(_F), (0, pad), mode='edge')
                      .reshape(_NF, 1, _F_BLK))

    T = np.linspace(0.1, 4.0, _TN, dtype=np.float32)
    tcol = jnp.asarray(T.reshape(_TN, 1))

    fr4 = frame.reshape(B, _J // _JB, _JB, frame.shape[-1])

    def s11(v):
        return jnp.asarray(v, jnp.float32).reshape(1, 1)

    out = pl.pallas_call(
        _fwd_kernel,
        grid=(B, _J // _JB, _NF),
        in_specs=[
            pl.BlockSpec((1, 1, _JB, 12), lambda b, j, f: (b, j, 0, 0)),
            pl.BlockSpec((1, 1, _F_BLK), lambda b, j, f: (f, 0, 0)),
            pl.BlockSpec((1, 1, _F_BLK), lambda b, j, f: (f, 0, 0)),
            pl.BlockSpec((_TN, 1), lambda b, j, f: (0, 0)),
            pl.BlockSpec((1, 1), lambda b, j, f: (0, 0)),
            pl.BlockSpec((1, 1), lambda b, j, f: (0, 0)),
            pl.BlockSpec((1, 1), lambda b, j, f: (0, 0)),
            pl.BlockSpec((1, 1), lambda b, j, f: (0, 0)),
        ],
        out_specs=pl.BlockSpec((1, _JB, _TN, _F_BLK),
                               lambda b, j, f: (b, j, 0, f)),
        out_shape=jax.ShapeDtypeStruct((B, _J, _TN, _F), jnp.float32),
    )(fr4, flx, fly, tcol,
      s11(tti_sigma), s11(a_max), s11(s_max), s11(reax_t))

    return out.transpose(0, 3, 2, 1)


# speedup vs baseline: 1.2280x; 1.0422x over previous
import jax
import jax.numpy as jnp
import numpy as np
from jax.experimental import pallas as pl

_F = 6600
_J = 22
_TN = 40
_F_BLK = 3328
_NF = 2


def _fwd_kernel(fr_ref, flx_ref, fly_ref, t_ref,
                sig_ref, amax_ref, smax_ref, reax_ref, out_ref):
    fr = fr_ref[0]
    x = fr[:, 1:2]
    y = fr[:, 2:3]
    vx = fr[:, 3:4]
    vy = fr[:, 4:5]
    ax = fr[:, 5:6]
    ay = fr[:, 6:7]

    sigma = sig_ref[0, 0]
    a_max = amax_ref[0, 0]
    s_max = smax_ref[0, 0]
    reax_t = reax_ref[0, 0]

    vxr = ax * reax_t + vx
    vyr = ay * reax_t + vy
    xr = x + vx * reax_t + 0.5 * ax * reax_t * reax_t
    yr = y + vy * reax_t + 0.5 * ay * reax_t * reax_t

    flx = flx_ref[0]
    fly = fly_ref[0]

    dx = flx - xr
    dy = fly - yr
    dmag = jnp.sqrt(dx * dx + dy * dy)
    s0 = jnp.clip((dx * vxr + dy * vyr) / dmag, -s_max, s_max)
    t_lt = (s_max - s0) / a_max
    d_lt = t_lt * (s0 + s_max) * 0.5
    soa = s0 / a_max
    t_lt = jnp.where(d_lt > dmag,
                     -soa + jnp.sqrt(soa * soa + 2.0 * dmag / a_max),
                     t_lt)
    d_lt = jnp.maximum(jnp.minimum(d_lt, dmag), 0.0)
    t_tot = reax_t + t_lt + (dmag - d_lt) / s_max

    kk = (jnp.pi / jnp.sqrt(3.0)) / sigma
    ez = jnp.exp(kk * t_tot)
    ct = jnp.exp(-kk * t_ref[...])

    w = ct.reshape(1, _TN, 1) * ez.reshape(_J, 1, _F_BLK) + 1.0
    out_ref[0] = 1.0 / w


def kernel(frame, tti_sigma, a_max, s_max, reax_t):
    B = frame.shape[0]

    x = np.linspace(0.5, 119.5, 120, dtype=np.float32)
    y = np.linspace(-0.5, 53.5, 55, dtype=np.float32)
    y[0] = -0.2
    yy, xx = np.meshgrid(y, x, indexing='ij')
    pad = _NF * _F_BLK - _F
    flx = jnp.asarray(np.pad(xx.reshape(_F), (0, pad), mode='edge')
                      .reshape(_NF, 1, _F_BLK))
    fly = jnp.asarray(np.pad(yy.reshape(_F), (0, pad), mode='edge')
                      .reshape(_NF, 1, _F_BLK))

    T = np.linspace(0.1, 4.0, _TN, dtype=np.float32)
    tcol = jnp.asarray(T.reshape(_TN, 1))

    def s11(v):
        return jnp.asarray(v, jnp.float32).reshape(1, 1)

    out = pl.pallas_call(
        _fwd_kernel,
        grid=(B, _NF),
        in_specs=[
            pl.BlockSpec((1, _J, 12), lambda b, f: (b, 0, 0)),
            pl.BlockSpec((1, 1, _F_BLK), lambda b, f: (f, 0, 0)),
            pl.BlockSpec((1, 1, _F_BLK), lambda b, f: (f, 0, 0)),
            pl.BlockSpec((_TN, 1), lambda b, f: (0, 0)),
            pl.BlockSpec((1, 1), lambda b, f: (0, 0)),
            pl.BlockSpec((1, 1), lambda b, f: (0, 0)),
            pl.BlockSpec((1, 1), lambda b, f: (0, 0)),
            pl.BlockSpec((1, 1), lambda b, f: (0, 0)),
        ],
        out_specs=pl.BlockSpec((1, _J, _TN, _F_BLK), lambda b, f: (b, 0, 0, f)),
        out_shape=jax.ShapeDtypeStruct((B, _J, _TN, _F), jnp.float32),
    )(frame, flx, fly, tcol,
      s11(tti_sigma), s11(a_max), s11(s_max), s11(reax_t))

    return out.transpose(0, 3, 2, 1)
